# Initial kernel scaffold; baseline (speedup 1.0000x reference)
import jax
import jax.numpy as jnp
from jax.experimental import pallas as pl


def kernel(x, edge_index, batch, W1, b1, W_lin1, b_lin1, W_lin2, b_lin2):
    raise NotImplementedError("write your pallas kernel here")


# speedup vs baseline: 22.3018x; 22.3018x over previous
import functools

import jax
import jax.numpy as jnp
from jax import lax
from jax.experimental import pallas as pl
from jax.experimental.pallas import tpu as pltpu
from jax.experimental.pallas import tpu_sc as plsc

_N = 10000
_E = 320000
_D = 128
_H = 64
_G = 64

_NP = 10240
_EP = 327680
_EROWS = _EP // 128
_ROWS_PER_TILE = _EROWS // 32
_CHUNK_ROWS = 8
_CHUNKS = _ROWS_PER_TILE // _CHUNK_ROWS
_RPT = _NP // 16
_RB = 1280
_NB = _NP // _RB


def _sc_degree(dst_rows, zeros16, ones16):
    mesh = plsc.VectorSubcoreMesh(core_axis_name="c", subcore_axis_name="s")

    @functools.partial(
        pl.kernel,
        mesh=mesh,
        out_type=jax.ShapeDtypeStruct((2 * _NP, 16), jnp.float32),
        scratch_types=[
            pltpu.VMEM((_CHUNK_ROWS, 128), jnp.int32),
            pltpu.VMEM((128, 16), jnp.float32),
            pltpu.VMEM_SHARED((_NP, 16), jnp.float32),
        ],
        compiler_params=pltpu.CompilerParams(use_tc_tiling_on_sc=False),
    )
    def k(dst_hbm, zeros_hbm, ones_hbm, out_hbm, idx_d, ones_v, acc):
        c = lax.axis_index("c")
        s = lax.axis_index("s")
        tid = c * 16 + s
        pltpu.sync_copy(zeros_hbm.at[pl.ds(s * _RPT, _RPT)], acc.at[pl.ds(s * _RPT, _RPT)])
        pltpu.sync_copy(ones_hbm, ones_v)
        plsc.subcore_barrier()

        def body(ci, carry):
            rb = tid * _ROWS_PER_TILE + ci * _CHUNK_ROWS
            pltpu.sync_copy(dst_hbm.at[pl.ds(rb, _CHUNK_ROWS)], idx_d)
            for j in range(_CHUNK_ROWS):
                pltpu.sync_copy(ones_v, acc.at[idx_d.at[j]], add=True)
            return carry

        lax.fori_loop(0, _CHUNKS, body, 0)
        plsc.subcore_barrier()
        pltpu.sync_copy(acc.at[pl.ds(s * _RPT, _RPT)],
                        out_hbm.at[pl.ds(c * _NP + s * _RPT, _RPT)])

    return k(dst_rows, zeros16, ones16)


def _sc_messages(g, src_rows, dst_rows, zeros64):
    mesh = plsc.VectorSubcoreMesh(core_axis_name="c", subcore_axis_name="s")

    @functools.partial(
        pl.kernel,
        mesh=mesh,
        out_type=jax.ShapeDtypeStruct((2 * _NP, _H), jnp.float32),
        scratch_types=[
            pltpu.VMEM((_CHUNK_ROWS, 128), jnp.int32),
            pltpu.VMEM((_CHUNK_ROWS, 128), jnp.int32),
            pltpu.VMEM((_CHUNK_ROWS * 128, _H), jnp.float32),
            pltpu.VMEM_SHARED((_NP, _H), jnp.float32),
            pltpu.SemaphoreType.DMA,
        ],
        compiler_params=pltpu.CompilerParams(use_tc_tiling_on_sc=False),
    )
    def k(g_hbm, src_hbm, dst_hbm, zeros_hbm, out_hbm, idx_s, idx_d, rows, acc, sem):
        c = lax.axis_index("c")
        s = lax.axis_index("s")
        tid = c * 16 + s
        pltpu.sync_copy(zeros_hbm.at[pl.ds(s * _RPT, _RPT)], acc.at[pl.ds(s * _RPT, _RPT)])
        plsc.subcore_barrier()

        def body(ci, carry):
            rb = tid * _ROWS_PER_TILE + ci * _CHUNK_ROWS
            pltpu.sync_copy(src_hbm.at[pl.ds(rb, _CHUNK_ROWS)], idx_s)
            pltpu.sync_copy(dst_hbm.at[pl.ds(rb, _CHUNK_ROWS)], idx_d)
            cps = [
                pltpu.async_copy(g_hbm.at[idx_s.at[j]],
                                 rows.at[pl.ds(j * 128, 128)], sem)
                for j in range(_CHUNK_ROWS)
            ]
            for cp in cps:
                cp.wait()
            for j in range(_CHUNK_ROWS):
                pltpu.sync_copy(rows.at[pl.ds(j * 128, 128)],
                                acc.at[idx_d.at[j]], add=True)
            return carry

        lax.fori_loop(0, _CHUNKS, body, 0)
        plsc.subcore_barrier()
        pltpu.sync_copy(acc.at[pl.ds(s * _RPT, _RPT)],
                        out_hbm.at[pl.ds(c * _NP + s * _RPT, _RPT)])

    return k(g, src_rows, dst_rows, zeros64)


def _tc_g(xp, W1, degp):
    def body(x_ref, w_ref, dp_ref, g_ref):
        deg = dp_ref[0, :, 0:1] + dp_ref[1, :, 0:1] + 1.0
        dinv = lax.rsqrt(deg)
        h = jnp.dot(x_ref[...], w_ref[...], preferred_element_type=jnp.float32)
        g_ref[...] = dinv * h

    return pl.pallas_call(
        body,
        grid=(_NB,),
        in_specs=[
            pl.BlockSpec((_RB, _D), lambda i: (i, 0)),
            pl.BlockSpec((_D, _H), lambda i: (0, 0)),
            pl.BlockSpec((2, _RB, 16), lambda i: (0, i, 0)),
        ],
        out_specs=pl.BlockSpec((_RB, _H), lambda i: (i, 0)),
        out_shape=jax.ShapeDtypeStruct((_NP, _H), jnp.float32),
    )(xp, W1, degp)


def _tc_final(accp, g, degp, batch3, b1r, Wl1, bl1r, Wl2, bl2r):
    def body(acc_ref, g_ref, dp_ref, b_ref, b1_ref, w1_ref, bl1_ref, w2_ref,
             bl2_ref, out_ref, u_acc):
        i = pl.program_id(0)

        @pl.when(i == 0)
        def _():
            u_acc[...] = jnp.zeros_like(u_acc)

        deg = dp_ref[0, :, 0:1] + dp_ref[1, :, 0:1] + 1.0
        dinv = lax.rsqrt(deg)
        hrelu = jnp.maximum(
            dinv * (acc_ref[0] + acc_ref[1] + g_ref[...]) + b1_ref[...], 0.0)
        bvec = b_ref[0, 0, :]
        onehot = (bvec[None, :] == lax.broadcasted_iota(jnp.int32, (_G, _RB), 0)
                  ).astype(jnp.float32)
        u_acc[...] += jnp.dot(onehot, hrelu, preferred_element_type=jnp.float32)

        @pl.when(i == _NB - 1)
        def _():
            u = u_acc[...]
            t = jnp.maximum(
                jnp.dot(u, w1_ref[...], preferred_element_type=jnp.float32)
                + bl1_ref[...], 0.0)
            out_ref[...] = (jnp.dot(t, w2_ref[...],
                                    preferred_element_type=jnp.float32)
                            + bl2_ref[...])

    return pl.pallas_call(
        body,
        grid=(_NB,),
        in_specs=[
            pl.BlockSpec((2, _RB, _H), lambda i: (0, i, 0)),
            pl.BlockSpec((_RB, _H), lambda i: (i, 0)),
            pl.BlockSpec((2, _RB, 16), lambda i: (0, i, 0)),
            pl.BlockSpec((1, 1, _RB), lambda i: (i, 0, 0)),
            pl.BlockSpec((1, _H), lambda i: (0, 0)),
            pl.BlockSpec((_H, 16), lambda i: (0, 0)),
            pl.BlockSpec((1, 16), lambda i: (0, 0)),
            pl.BlockSpec((16, 1), lambda i: (0, 0)),
            pl.BlockSpec((1, 1), lambda i: (0, 0)),
        ],
        out_specs=pl.BlockSpec((_G, 1), lambda i: (0, 0)),
        out_shape=jax.ShapeDtypeStruct((_G, 1), jnp.float32),
        scratch_shapes=[pltpu.VMEM((_G, _G), jnp.float32)],
    )(accp, g, degp, batch3, b1r, Wl1, bl1r, Wl2, bl2r)


def kernel(x, edge_index, batch, W1, b1, W_lin1, b_lin1, W_lin2, b_lin2):
    src = edge_index[0]
    dst = edge_index[1]
    pad_e = _EP - _E
    srcp = jnp.concatenate(
        [src, jnp.full((pad_e,), _N, jnp.int32)]).reshape(_EROWS, 128)
    dstp = jnp.concatenate(
        [dst, jnp.full((pad_e,), _N, jnp.int32)]).reshape(_EROWS, 128)
    xp = jnp.zeros((_NP, _D), jnp.float32).at[:_N].set(x)
    zeros16 = jnp.zeros((_NP, 16), jnp.float32)
    ones16 = jnp.ones((128, 16), jnp.float32)
    zeros64 = jnp.zeros((_NP, _H), jnp.float32)

    degp = _sc_degree(dstp, zeros16, ones16).reshape(2, _NP, 16)
    g = _tc_g(xp, W1, degp)
    accp = _sc_messages(g, srcp, dstp, zeros64).reshape(2, _NP, _H)

    batchp = jnp.concatenate(
        [batch, jnp.full((_NP - _N,), _G, jnp.int32)]).reshape(_NB, 1, _RB)
    out = _tc_final(accp, g, degp, batchp,
                    b1.reshape(1, _H), W_lin1, b_lin1.reshape(1, 16),
                    W_lin2, b_lin2.reshape(1, 1))
    return out.reshape(-1)


# speedup vs baseline: 25.0375x; 1.1227x over previous
import functools

import jax
import jax.numpy as jnp
from jax import lax
from jax.experimental import pallas as pl
from jax.experimental.pallas import tpu as pltpu
from jax.experimental.pallas import tpu_sc as plsc

_N = 10000
_E = 320000
_D = 128
_H = 64
_G = 64

_NP = 10240
_EP = 327680
_EROWS = _EP // 128
_ROWS_PER_TILE = _EROWS // 32
_CHUNK_ROWS = 8
_CHUNKS = _ROWS_PER_TILE // _CHUNK_ROWS
_RPT = _NP // 16
_RB = 1280
_NB = _NP // _RB


def _sc_degree(dst_rows, zeros16, ones16):
    mesh = plsc.VectorSubcoreMesh(core_axis_name="c", subcore_axis_name="s")

    @functools.partial(
        pl.kernel,
        mesh=mesh,
        out_type=jax.ShapeDtypeStruct((2 * _NP, 16), jnp.float32),
        scratch_types=[
            pltpu.VMEM((_CHUNK_ROWS, 128), jnp.int32),
            pltpu.VMEM((128, 16), jnp.float32),
            pltpu.VMEM_SHARED((_NP, 16), jnp.float32),
        ],
        compiler_params=pltpu.CompilerParams(use_tc_tiling_on_sc=False),
    )
    def k(dst_hbm, zeros_hbm, ones_hbm, out_hbm, idx_d, ones_v, acc):
        c = lax.axis_index("c")
        s = lax.axis_index("s")
        tid = c * 16 + s
        pltpu.sync_copy(zeros_hbm.at[pl.ds(s * _RPT, _RPT)], acc.at[pl.ds(s * _RPT, _RPT)])
        pltpu.sync_copy(ones_hbm, ones_v)
        plsc.subcore_barrier()

        def body(ci, carry):
            rb = tid * _ROWS_PER_TILE + ci * _CHUNK_ROWS
            pltpu.sync_copy(dst_hbm.at[pl.ds(rb, _CHUNK_ROWS)], idx_d)
            for j in range(_CHUNK_ROWS):
                pltpu.sync_copy(ones_v, acc.at[idx_d.at[j]], add=True)
            return carry

        lax.fori_loop(0, _CHUNKS, body, 0)
        plsc.subcore_barrier()
        pltpu.sync_copy(acc.at[pl.ds(s * _RPT, _RPT)],
                        out_hbm.at[pl.ds(c * _NP + s * _RPT, _RPT)])

    return k(dst_rows, zeros16, ones16)


_CR = 4
_CE = _CR * 128
_R0 = 120
_R1 = (2 * _ROWS_PER_TILE) - _R0
_NPAIR0 = _R0 // (2 * _CR)
_NPAIR1 = _R1 // (2 * _CR)


def _sc_messages(g, src_rows, dst_rows, zeros64):
    mesh = plsc.VectorSubcoreMesh(core_axis_name="c", subcore_axis_name="s")

    @functools.partial(
        pl.kernel,
        mesh=mesh,
        out_type=jax.ShapeDtypeStruct((2 * _NP, _H), jnp.float32),
        scratch_types=[
            pltpu.VMEM((2 * _CR, 128), jnp.int32),
            pltpu.VMEM((2 * _CR, 128), jnp.int32),
            pltpu.VMEM((2 * _CE, _H), jnp.float32),
            pltpu.VMEM_SHARED((_NP, _H), jnp.float32),
            pltpu.SemaphoreType.DMA,
            pltpu.SemaphoreType.DMA,
            pltpu.SemaphoreType.DMA,
            pltpu.SemaphoreType.DMA,
        ],
        compiler_params=pltpu.CompilerParams(use_tc_tiling_on_sc=False),
    )
    def k(g_hbm, src_hbm, dst_hbm, zeros_hbm, out_hbm, idx_s, idx_d, rows,
          acc, gsem0, gsem1, ssem0, ssem1):
        c = lax.axis_index("c")
        s = lax.axis_index("s")
        base = jnp.where(c == 0, s * _R0, 16 * _R0 + s * _R1)
        n_pair = jnp.where(c == 0, _NPAIR0, _NPAIR1)
        pltpu.sync_copy(zeros_hbm.at[pl.ds(s * _RPT, _RPT)], acc.at[pl.ds(s * _RPT, _RPT)])
        plsc.subcore_barrier()

        def copy_idx(ch, b):
            rb = base + ch * _CR
            pltpu.sync_copy(src_hbm.at[pl.ds(rb, _CR)], idx_s.at[pl.ds(b * _CR, _CR)])
            pltpu.sync_copy(dst_hbm.at[pl.ds(rb, _CR)], idx_d.at[pl.ds(b * _CR, _CR)])

        def fire_gathers(b, gsem):
            for j in range(_CR):
                pltpu.async_copy(g_hbm.at[idx_s.at[b * _CR + j]],
                                 rows.at[pl.ds(b * _CE + j * 128, 128)], gsem)

        def wait_gathers(b, gsem):
            pltpu.make_async_copy(g_hbm.at[pl.ds(0, _CE)],
                                  rows.at[pl.ds(b * _CE, _CE)], gsem).wait()

        def fire_scatters(b, ssem):
            for j in range(_CR):
                pltpu.async_copy(rows.at[pl.ds(b * _CE + j * 128, 128)],
                                 acc.at[idx_d.at[b * _CR + j]], ssem, add=True)

        def wait_scatters(b, ssem):
            pltpu.make_async_copy(g_hbm.at[pl.ds(0, _CE)],
                                  rows.at[pl.ds(b * _CE, _CE)], ssem).wait()

        copy_idx(0, 0)
        fire_gathers(0, gsem0)

        def pair(i, carry):
            @pl.when(i >= 1)
            def _():
                wait_scatters(1, ssem1)
            copy_idx(2 * i + 1, 1)
            fire_gathers(1, gsem1)
            wait_gathers(0, gsem0)
            fire_scatters(0, ssem0)

            @pl.when(i + 1 < n_pair)
            def _():
                wait_scatters(0, ssem0)
                copy_idx(2 * i + 2, 0)
                fire_gathers(0, gsem0)
            wait_gathers(1, gsem1)
            fire_scatters(1, ssem1)
            return carry

        lax.fori_loop(0, n_pair, pair, 0)
        wait_scatters(0, ssem0)
        wait_scatters(1, ssem1)
        plsc.subcore_barrier()
        pltpu.sync_copy(acc.at[pl.ds(s * _RPT, _RPT)],
                        out_hbm.at[pl.ds(c * _NP + s * _RPT, _RPT)])

    return k(g, src_rows, dst_rows, zeros64)


def _tc_g(xp, W1, degp):
    def body(x_ref, w_ref, d0_ref, d1_ref, g_ref):
        deg = d0_ref[:, 0:1] + d1_ref[:, 0:1] + 1.0
        dinv = lax.rsqrt(deg)
        h = jnp.dot(x_ref[...], w_ref[...], preferred_element_type=jnp.float32)
        g_ref[...] = dinv * h

    return pl.pallas_call(
        body,
        grid=(_NB,),
        in_specs=[
            pl.BlockSpec((_RB, _D), lambda i: (i, 0)),
            pl.BlockSpec((_D, _H), lambda i: (0, 0)),
            pl.BlockSpec((_RB, 16), lambda i: (i, 0)),
            pl.BlockSpec((_RB, 16), lambda i: (_NB + i, 0)),
        ],
        out_specs=pl.BlockSpec((_RB, _H), lambda i: (i, 0)),
        out_shape=jax.ShapeDtypeStruct((_NP, _H), jnp.float32),
    )(xp, W1, degp, degp)


def _tc_final(accp, g, degp, batch3, b1r, Wl1, bl1r, Wl2, bl2r):
    def body(a0_ref, a1_ref, g_ref, d0_ref, d1_ref, b_ref, b1_ref, w1_ref,
             bl1_ref, w2_ref, bl2_ref, out_ref, u_acc):
        i = pl.program_id(0)

        @pl.when(i == 0)
        def _():
            u_acc[...] = jnp.zeros_like(u_acc)

        deg = d0_ref[:, 0:1] + d1_ref[:, 0:1] + 1.0
        dinv = lax.rsqrt(deg)
        hrelu = jnp.maximum(
            dinv * (a0_ref[...] + a1_ref[...] + g_ref[...]) + b1_ref[...], 0.0)
        bvec = b_ref[0, 0, :]
        onehot = (bvec[None, :] == lax.broadcasted_iota(jnp.int32, (_G, _RB), 0)
                  ).astype(jnp.float32)
        u_acc[...] += jnp.dot(onehot, hrelu, preferred_element_type=jnp.float32)

        @pl.when(i == _NB - 1)
        def _():
            u = u_acc[...]
            t = jnp.maximum(
                jnp.dot(u, w1_ref[...], preferred_element_type=jnp.float32)
                + bl1_ref[...], 0.0)
            out_ref[...] = (jnp.dot(t, w2_ref[...],
                                    preferred_element_type=jnp.float32)
                            + bl2_ref[...])

    return pl.pallas_call(
        body,
        grid=(_NB,),
        in_specs=[
            pl.BlockSpec((_RB, _H), lambda i: (i, 0)),
            pl.BlockSpec((_RB, _H), lambda i: (_NB + i, 0)),
            pl.BlockSpec((_RB, _H), lambda i: (i, 0)),
            pl.BlockSpec((_RB, 16), lambda i: (i, 0)),
            pl.BlockSpec((_RB, 16), lambda i: (_NB + i, 0)),
            pl.BlockSpec((1, 1, _RB), lambda i: (i, 0, 0)),
            pl.BlockSpec((1, _H), lambda i: (0, 0)),
            pl.BlockSpec((_H, 16), lambda i: (0, 0)),
            pl.BlockSpec((1, 16), lambda i: (0, 0)),
            pl.BlockSpec((16, 1), lambda i: (0, 0)),
            pl.BlockSpec((1, 1), lambda i: (0, 0)),
        ],
        out_specs=pl.BlockSpec((_G, 1), lambda i: (0, 0)),
        out_shape=jax.ShapeDtypeStruct((_G, 1), jnp.float32),
        scratch_shapes=[pltpu.VMEM((_G, _G), jnp.float32)],
    )(accp, accp, g, degp, degp, batch3, b1r, Wl1, bl1r, Wl2, bl2r)


def kernel(x, edge_index, batch, W1, b1, W_lin1, b_lin1, W_lin2, b_lin2):
    src = edge_index[0]
    dst = edge_index[1]
    pad_e = _EP - _E
    srcp = jnp.concatenate(
        [src, jnp.full((pad_e,), _N, jnp.int32)]).reshape(_EROWS, 128)
    dstp = jnp.concatenate(
        [dst, jnp.full((pad_e,), _N, jnp.int32)]).reshape(_EROWS, 128)
    xp = jnp.zeros((_NP, _D), jnp.float32).at[:_N].set(x)
    zeros16 = jnp.zeros((_NP, 16), jnp.float32)
    ones16 = jnp.ones((128, 16), jnp.float32)
    zeros64 = jnp.zeros((_NP, _H), jnp.float32)

    degp = _sc_degree(dstp, zeros16, ones16)
    g = _tc_g(xp, W1, degp)
    accp = _sc_messages(g, srcp, dstp, zeros64)

    batchp = jnp.concatenate(
        [batch, jnp.full((_NP - _N,), _G, jnp.int32)]).reshape(_NB, 1, _RB)
    out = _tc_final(accp, g, degp, batchp,
                    b1.reshape(1, _H), W_lin1, b_lin1.reshape(1, 16),
                    W_lin2, b_lin2.reshape(1, 1))
    return out.reshape(-1)
